# baseline (speedup 1.0000x reference)
import functools

import jax
import jax.numpy as jnp
import numpy as np
from jax import lax
from jax.experimental import pallas as pl
from jax.experimental.pallas import tpu as pltpu
from jax.experimental.pallas import tpu_sc as plsc

N_S = 10000
N_R = 10000
E = 320000
DIN = 128
DH = 128
H = 8
D = 16
NEG_SLOPE = 0.2

NW = 32
EPW = E // NW
CH = 40
NCHUNK = EPW // CH
NP = 10240
RPT = NP // 16
BLK = 400

_DNUMS = lax.GatherDimensionNumbers(
    offset_dims=(), collapsed_slice_dims=(0,), start_index_map=(0,))


def _proj_body(x_ref, w_ref, b_ref, a_ref, h_ref, ha_ref):
    h = jnp.dot(x_ref[0], w_ref[0], preferred_element_type=jnp.float32)
    h = h + b_ref[0]
    h_ref[0] = h
    ha_ref[0] = jnp.dot(h, a_ref[0], preferred_element_type=jnp.float32)


def _proj(xs, Ws, bs, As):
    n = xs.shape[1]
    return pl.pallas_call(
        _proj_body,
        grid=(2, n // BLK),
        in_specs=[
            pl.BlockSpec((1, BLK, DIN), lambda t, i: (t, i, 0)),
            pl.BlockSpec((1, DIN, DH), lambda t, i: (t, 0, 0)),
            pl.BlockSpec((1, 1, DH), lambda t, i: (t, 0, 0)),
            pl.BlockSpec((1, DH, 128), lambda t, i: (t, 0, 0)),
        ],
        out_specs=[
            pl.BlockSpec((1, BLK, DH), lambda t, i: (t, i, 0)),
            pl.BlockSpec((1, BLK, 128), lambda t, i: (t, i, 0)),
        ],
        out_shape=[
            jax.ShapeDtypeStruct((2, n, DH), jnp.float32),
            jax.ShapeDtypeStruct((2, n, 128), jnp.float32),
        ],
    )(xs, Ws, bs, As)


def _edge_sc_body(src_hbm, dst_hbm, asrc_hbm, adst_hbm, h_hbm,
                  psum_hbm, pden_hbm,
                  sidx, didx, as_v, ad_v, h_v, ex_v, msg_v,
                  out_sh, den_sh, sem_i, sem_g, sem_s):
    c = lax.axis_index("c")
    s = lax.axis_index("s")
    wid = s * 2 + c
    base = wid * EPW

    zeros16 = jnp.zeros((16,), jnp.float32)
    lanes = lax.iota(jnp.int32, 16)
    mask = jnp.where(lanes < 8, 1.0, 0.0)
    idx_h = [(lanes * 0 + h).reshape(16, 1) for h in range(8)]

    def zrow(e, carry):
        for j in range(8):
            msg_v[e, pl.ds(16 * j, 16)] = zeros16
        ex_v[e, :] = zeros16
        return carry

    lax.fori_loop(0, CH, zrow, 0)

    def zblk(i, carry):
        off = s * RPT + i * CH
        pltpu.sync_copy(msg_v.at[pl.ds(0, CH)], out_sh.at[pl.ds(off, CH)])
        pltpu.sync_copy(ex_v.at[pl.ds(0, CH)], den_sh.at[pl.ds(off, CH)])
        return carry

    lax.fori_loop(0, RPT // CH, zblk, 0)
    plsc.subcore_barrier()

    def issue_idx(k):
        q = lax.rem(k, 4)
        b = base + k * CH
        pltpu.async_copy(src_hbm.at[pl.ds(b, CH)], sidx.at[q], sem_i)
        pltpu.async_copy(dst_hbm.at[pl.ds(b, CH)], didx.at[q], sem_i)

    def wait_idx():
        pltpu.make_async_copy(
            src_hbm.at[pl.ds(0, CH)], sidx.at[0], sem_i).wait()
        pltpu.make_async_copy(
            dst_hbm.at[pl.ds(0, CH)], didx.at[0], sem_i).wait()

    def issue_gathers(k):
        q = lax.rem(k, 4)
        p = lax.rem(k, 2)
        o = p * CH
        pltpu.async_copy(asrc_hbm.at[sidx.at[q]],
                         as_v.at[pl.ds(o, CH)], sem_g)
        pltpu.async_copy(adst_hbm.at[didx.at[q]],
                         ad_v.at[pl.ds(o, CH)], sem_g)
        pltpu.async_copy(h_hbm.at[sidx.at[q]],
                         h_v.at[pl.ds(o, CH)], sem_g)

    def wait_gathers():
        pltpu.make_async_copy(
            asrc_hbm.at[pl.ds(0, CH)], as_v.at[pl.ds(0, CH)], sem_g).wait()
        pltpu.make_async_copy(
            adst_hbm.at[pl.ds(0, CH)], ad_v.at[pl.ds(0, CH)], sem_g).wait()
        pltpu.make_async_copy(
            h_hbm.at[pl.ds(0, CH)], h_v.at[pl.ds(0, CH)], sem_g).wait()

    def wait_scatters():
        pltpu.make_async_copy(
            asrc_hbm.at[pl.ds(0, CH)], ex_v.at[pl.ds(0, CH)], sem_s).wait()
        pltpu.make_async_copy(
            h_hbm.at[pl.ds(0, CH)], msg_v.at[pl.ds(0, CH)], sem_s).wait()

    issue_idx(0)
    wait_idx()
    issue_gathers(0)
    issue_idx(1)

    def chunk(k, carry):
        p = lax.rem(k, 2)
        q = lax.rem(k, 4)
        o = p * CH
        wait_gathers()

        @pl.when(k >= 2)
        def _():
            wait_scatters()

        @pl.when(k + 1 < NCHUNK)
        def _():
            wait_idx()
            issue_gathers(k + 1)

        @pl.when(k + 2 < NCHUNK)
        def _():
            issue_idx(k + 2)

        @plsc.parallel_loop(0, CH, 1, unroll=4)
        def edge(e):
            eo = o + e
            alpha = as_v[eo, :] + ad_v[eo, :]
            alpha = jnp.where(alpha > 0, alpha, NEG_SLOPE * alpha)
            ex = jnp.exp(alpha) * mask
            ex_v[eo, :] = ex
            for h in range(8):
                spl = lax.gather(ex, idx_h[h], _DNUMS, (1,),
                                 mode=lax.GatherScatterMode.PROMISE_IN_BOUNDS)
                msg_v[eo, pl.ds(16 * h, 16)] = (
                    h_v[eo, pl.ds(16 * h, 16)] * spl)
        pltpu.async_copy(ex_v.at[pl.ds(o, CH)],
                         den_sh.at[didx.at[q]], sem_s, add=True)
        pltpu.async_copy(msg_v.at[pl.ds(o, CH)],
                         out_sh.at[didx.at[q]], sem_s, add=True)
        return carry

    lax.fori_loop(0, NCHUNK, chunk, 0)
    wait_scatters()
    wait_scatters()
    plsc.subcore_barrier()

    off = c * NP + s * RPT
    pltpu.sync_copy(out_sh.at[pl.ds(s * RPT, RPT)], psum_hbm.at[pl.ds(off, RPT)])
    pltpu.sync_copy(den_sh.at[pl.ds(s * RPT, RPT)], pden_hbm.at[pl.ds(off, RPT)])


def _edge_sc(src, dst, asrc, adst, h):
    mesh = plsc.VectorSubcoreMesh(core_axis_name="c", subcore_axis_name="s")
    f = pl.kernel(
        _edge_sc_body,
        out_type=[
            jax.ShapeDtypeStruct((2 * NP, 128), jnp.float32),
            jax.ShapeDtypeStruct((2 * NP, 16), jnp.float32),
        ],
        mesh=mesh,
        compiler_params=pltpu.CompilerParams(
            needs_layout_passes=False, use_tc_tiling_on_sc=False),
        scratch_types=[
            pltpu.VMEM((4, CH), jnp.int32),
            pltpu.VMEM((4, CH), jnp.int32),
            pltpu.VMEM((2 * CH, 16), jnp.float32),
            pltpu.VMEM((2 * CH, 16), jnp.float32),
            pltpu.VMEM((2 * CH, 128), jnp.float32),
            pltpu.VMEM((2 * CH, 16), jnp.float32),
            pltpu.VMEM((2 * CH, 128), jnp.float32),
            pltpu.VMEM_SHARED((NP, 128), jnp.float32),
            pltpu.VMEM_SHARED((NP, 16), jnp.float32),
            pltpu.SemaphoreType.DMA,
            pltpu.SemaphoreType.DMA,
            pltpu.SemaphoreType.DMA,
        ],
    )
    return f(src, dst, asrc, adst, h)


def _comb_body(p0_ref, p1_ref, d0_ref, d1_ref, r16_ref, wk_ref, bk_ref, q_ref,
               out_ref, s_ref):
    den = jnp.dot(d0_ref[0] + d1_ref[0], r16_ref[...],
                  preferred_element_type=jnp.float32) + 1e-16
    out = jnp.maximum((p0_ref[0] + p1_ref[0]) / den, 0.0)
    out_ref[0] = out
    t = jnp.tanh(jnp.dot(out, wk_ref[...],
                         preferred_element_type=jnp.float32) + bk_ref[...])

    tt = pl.program_id(0)

    @pl.when(pl.program_id(1) == 0)
    def _():
        s_ref[tt, 0] = 0.0

    s_ref[tt, 0] += jnp.sum(t * q_ref[...])


def _comb(p0s, p1s, d0s, d1s, r16, wk, bk, qrow):
    return pl.pallas_call(
        _comb_body,
        grid=(2, N_S // BLK),
        in_specs=[
            pl.BlockSpec((1, BLK, 128), lambda t, i: (t, i, 0)),
            pl.BlockSpec((1, BLK, 128), lambda t, i: (t, i, 0)),
            pl.BlockSpec((1, BLK, 16), lambda t, i: (t, i, 0)),
            pl.BlockSpec((1, BLK, 16), lambda t, i: (t, i, 0)),
            pl.BlockSpec((16, 128), lambda t, i: (0, 0)),
            pl.BlockSpec((DH, DH), lambda t, i: (0, 0)),
            pl.BlockSpec((1, DH), lambda t, i: (0, 0)),
            pl.BlockSpec((1, DH), lambda t, i: (0, 0)),
        ],
        out_specs=[
            pl.BlockSpec((1, BLK, DH), lambda t, i: (t, i, 0)),
            pl.BlockSpec((2, 1), lambda t, i: (0, 0), memory_space=pltpu.SMEM),
        ],
        out_shape=[
            jax.ShapeDtypeStruct((2, N_S, DH), jnp.float32),
            jax.ShapeDtypeStruct((2, 1), jnp.float32),
        ],
    )(p0s, p1s, d0s, d1s, r16, wk, bk, qrow)


def _final_body(ss_ref, rs_ref, s0_ref, s1_ref, wout_ref, bout_ref, o_ref):
    d = (s1_ref[0, 0] - s0_ref[0, 0]) / N_S
    w0 = 1.0 / (1.0 + jnp.exp(jnp.full((1, 128), d, jnp.float32)))
    mix = ss_ref[...] * w0 + rs_ref[...] * (1.0 - w0)
    o_ref[...] = jnp.dot(mix, wout_ref[...],
                         preferred_element_type=jnp.float32) + bout_ref[...]


def _final(out_ss, out_rs, s_ss, s_rs, wout, bout):
    return pl.pallas_call(
        _final_body,
        grid=(N_S // BLK,),
        in_specs=[
            pl.BlockSpec((BLK, DH), lambda i: (i, 0)),
            pl.BlockSpec((BLK, DH), lambda i: (i, 0)),
            pl.BlockSpec(memory_space=pltpu.SMEM),
            pl.BlockSpec(memory_space=pltpu.SMEM),
            pl.BlockSpec((DH, 128), lambda i: (0, 0)),
            pl.BlockSpec((1, 128), lambda i: (0, 0)),
        ],
        out_specs=pl.BlockSpec((BLK, 128), lambda i: (i, 0)),
        out_shape=jax.ShapeDtypeStruct((N_S, 128), jnp.float32),
    )(out_ss, out_rs, s_ss, s_rs, wout, bout)


def _att_mat(att):
    rows = jnp.arange(DH)
    return jnp.zeros((DH, 16), jnp.float32).at[rows, rows // D].set(
        att.reshape(DH))


def kernel(x_SUBJECT, x_REGION, edge_index_SUBJECT__to__SUBJECT,
           edge_index_REGION__in__SUBJECT,
           W_proj_SUBJECT, b_proj_SUBJECT, W_proj_REGION, b_proj_REGION,
           att_src_S2S, att_dst_S2S, att_src_R2S, att_dst_R2S,
           W_k, b_k, q, W_out, b_out):
    ei_ss = edge_index_SUBJECT__to__SUBJECT
    ei_rs = edge_index_REGION__in__SUBJECT

    a_s = jnp.concatenate(
        [_att_mat(att_src_S2S), _att_mat(att_dst_S2S), _att_mat(att_dst_R2S),
         jnp.zeros((DH, 128 - 48), jnp.float32)], axis=1)
    a_r = jnp.concatenate(
        [_att_mat(att_src_R2S), jnp.zeros((DH, 128 - 16), jnp.float32)],
        axis=1)
    cols = jnp.arange(128)
    r16 = jnp.zeros((16, 128), jnp.float32).at[cols // D, cols].set(1.0)
    wout_pad = jnp.zeros((DH, 128), jnp.float32).at[:, :2].set(W_out)
    bout_pad = jnp.zeros((1, 128), jnp.float32).at[0, :2].set(b_out)

    xs = jnp.stack([x_SUBJECT, x_REGION])
    Ws = jnp.stack([W_proj_SUBJECT, W_proj_REGION])
    bs = jnp.stack([b_proj_SUBJECT.reshape(1, DH),
                    b_proj_REGION.reshape(1, DH)])
    As = jnp.stack([a_s, a_r])
    hs, has = _proj(xs, Ws, bs, As)

    psum_ss, pden_ss = _edge_sc(ei_ss[0], ei_ss[1],
                                has[0, :, 0:16], has[0, :, 16:32], hs[0])
    psum_rs, pden_rs = _edge_sc(ei_rs[0], ei_rs[1],
                                has[1, :, 0:16], has[0, :, 32:48], hs[1])

    p0s = jnp.stack([psum_ss[:N_S], psum_rs[:N_S]])
    p1s = jnp.stack([psum_ss[NP:NP + N_S], psum_rs[NP:NP + N_S]])
    d0s = jnp.stack([pden_ss[:N_S], pden_rs[:N_S]])
    d1s = jnp.stack([pden_ss[NP:NP + N_S], pden_rs[NP:NP + N_S]])
    outs, scores = _comb(p0s, p1s, d0s, d1s,
                         r16, W_k, b_k.reshape(1, DH), q.reshape(1, DH))

    return _final(outs[0], outs[1], scores[0:1], scores[1:2],
                  wout_pad, bout_pad)[:, :2]


# speedup vs baseline: 1.1059x; 1.1059x over previous
import functools

import jax
import jax.numpy as jnp
import numpy as np
from jax import lax
from jax.experimental import pallas as pl
from jax.experimental.pallas import tpu as pltpu
from jax.experimental.pallas import tpu_sc as plsc

N_S = 10000
N_R = 10000
E = 320000
DIN = 128
DH = 128
H = 8
D = 16
NEG_SLOPE = 0.2

NW = 32
EPW = E // NW
CH = 40
NCHUNK = EPW // CH
NP = 10240
RPT = NP // 16
BLK = 400

_DNUMS = lax.GatherDimensionNumbers(
    offset_dims=(), collapsed_slice_dims=(0,), start_index_map=(0,))


def _proj_body(x_ref, w_ref, b_ref, a_ref, h_ref, ha_ref):
    h = jnp.dot(x_ref[...], w_ref[...], preferred_element_type=jnp.float32)
    h = h + b_ref[...]
    h_ref[...] = h
    ha_ref[...] = jnp.dot(h, a_ref[...], preferred_element_type=jnp.float32)


def _proj(x, W, b, A):
    n = x.shape[0]
    return pl.pallas_call(
        _proj_body,
        grid=(n // BLK,),
        in_specs=[
            pl.BlockSpec((BLK, DIN), lambda i: (i, 0)),
            pl.BlockSpec((DIN, DH), lambda i: (0, 0)),
            pl.BlockSpec((1, DH), lambda i: (0, 0)),
            pl.BlockSpec((DH, 128), lambda i: (0, 0)),
        ],
        out_specs=[
            pl.BlockSpec((BLK, DH), lambda i: (i, 0)),
            pl.BlockSpec((BLK, 128), lambda i: (i, 0)),
        ],
        out_shape=[
            jax.ShapeDtypeStruct((n, DH), jnp.float32),
            jax.ShapeDtypeStruct((n, 128), jnp.float32),
        ],
    )(x, W, b.reshape(1, DH), A)


def _edge_sc_body(src_hbm, dst_hbm, asrc_hbm, adst_hbm, h_hbm,
                  psum_hbm, pden_hbm,
                  sidx, didx, as_v, ad_v, h_v, ex_v, msg_v,
                  out_sh, den_sh, sem_i, sem_g, sem_s):
    c = lax.axis_index("c")
    s = lax.axis_index("s")
    wid = s * 2 + c
    base = wid * EPW

    zeros16 = jnp.zeros((16,), jnp.float32)
    lanes = lax.iota(jnp.int32, 16)
    mask = jnp.where(lanes < 8, 1.0, 0.0)
    idx_h = [(lanes * 0 + h).reshape(16, 1) for h in range(8)]

    def zrow(e, carry):
        for j in range(8):
            msg_v[e, pl.ds(16 * j, 16)] = zeros16
        ex_v[e, :] = zeros16
        return carry

    lax.fori_loop(0, CH, zrow, 0)

    def zblk(i, carry):
        off = s * RPT + i * CH
        pltpu.sync_copy(msg_v.at[pl.ds(0, CH)], out_sh.at[pl.ds(off, CH)])
        pltpu.sync_copy(ex_v.at[pl.ds(0, CH)], den_sh.at[pl.ds(off, CH)])
        return carry

    lax.fori_loop(0, RPT // CH, zblk, 0)
    plsc.subcore_barrier()

    def issue_idx(k):
        q = lax.rem(k, 4)
        b = base + k * CH
        pltpu.async_copy(src_hbm.at[pl.ds(b, CH)], sidx.at[q], sem_i)
        pltpu.async_copy(dst_hbm.at[pl.ds(b, CH)], didx.at[q], sem_i)

    def wait_idx():
        pltpu.make_async_copy(
            src_hbm.at[pl.ds(0, CH)], sidx.at[0], sem_i).wait()
        pltpu.make_async_copy(
            dst_hbm.at[pl.ds(0, CH)], didx.at[0], sem_i).wait()

    def issue_gathers(k):
        q = lax.rem(k, 4)
        p = lax.rem(k, 2)
        o = p * CH
        pltpu.async_copy(asrc_hbm.at[sidx.at[q]],
                         as_v.at[pl.ds(o, CH)], sem_g)
        pltpu.async_copy(adst_hbm.at[didx.at[q]],
                         ad_v.at[pl.ds(o, CH)], sem_g)
        pltpu.async_copy(h_hbm.at[sidx.at[q]],
                         h_v.at[pl.ds(o, CH)], sem_g)

    def wait_gathers():
        pltpu.make_async_copy(
            asrc_hbm.at[pl.ds(0, CH)], as_v.at[pl.ds(0, CH)], sem_g).wait()
        pltpu.make_async_copy(
            adst_hbm.at[pl.ds(0, CH)], ad_v.at[pl.ds(0, CH)], sem_g).wait()
        pltpu.make_async_copy(
            h_hbm.at[pl.ds(0, CH)], h_v.at[pl.ds(0, CH)], sem_g).wait()

    def wait_scatters():
        pltpu.make_async_copy(
            asrc_hbm.at[pl.ds(0, CH)], ex_v.at[pl.ds(0, CH)], sem_s).wait()
        pltpu.make_async_copy(
            h_hbm.at[pl.ds(0, CH)], msg_v.at[pl.ds(0, CH)], sem_s).wait()

    issue_idx(0)
    wait_idx()
    issue_gathers(0)
    issue_idx(1)

    def chunk(k, carry):
        p = lax.rem(k, 2)
        q = lax.rem(k, 4)
        o = p * CH
        wait_gathers()

        @pl.when(k >= 2)
        def _():
            wait_scatters()

        @pl.when(k + 1 < NCHUNK)
        def _():
            wait_idx()
            issue_gathers(k + 1)

        @pl.when(k + 2 < NCHUNK)
        def _():
            issue_idx(k + 2)

        @plsc.parallel_loop(0, CH, 1, unroll=4)
        def edge(e):
            eo = o + e
            alpha = as_v[eo, :] + ad_v[eo, :]
            alpha = jnp.where(alpha > 0, alpha, NEG_SLOPE * alpha)
            ex = jnp.exp(alpha) * mask
            ex_v[eo, :] = ex
            for h in range(8):
                spl = lax.gather(ex, idx_h[h], _DNUMS, (1,),
                                 mode=lax.GatherScatterMode.PROMISE_IN_BOUNDS)
                msg_v[eo, pl.ds(16 * h, 16)] = (
                    h_v[eo, pl.ds(16 * h, 16)] * spl)
        pltpu.async_copy(ex_v.at[pl.ds(o, CH)],
                         den_sh.at[didx.at[q]], sem_s, add=True)
        pltpu.async_copy(msg_v.at[pl.ds(o, CH)],
                         out_sh.at[didx.at[q]], sem_s, add=True)
        return carry

    lax.fori_loop(0, NCHUNK, chunk, 0)
    wait_scatters()
    wait_scatters()
    plsc.subcore_barrier()

    off = c * NP + s * RPT
    pltpu.sync_copy(out_sh.at[pl.ds(s * RPT, RPT)], psum_hbm.at[pl.ds(off, RPT)])
    pltpu.sync_copy(den_sh.at[pl.ds(s * RPT, RPT)], pden_hbm.at[pl.ds(off, RPT)])


def _edge_sc(src, dst, asrc, adst, h):
    mesh = plsc.VectorSubcoreMesh(core_axis_name="c", subcore_axis_name="s")
    f = pl.kernel(
        _edge_sc_body,
        out_type=[
            jax.ShapeDtypeStruct((2 * NP, 128), jnp.float32),
            jax.ShapeDtypeStruct((2 * NP, 16), jnp.float32),
        ],
        mesh=mesh,
        compiler_params=pltpu.CompilerParams(
            needs_layout_passes=False, use_tc_tiling_on_sc=False),
        scratch_types=[
            pltpu.VMEM((4, CH), jnp.int32),
            pltpu.VMEM((4, CH), jnp.int32),
            pltpu.VMEM((2 * CH, 16), jnp.float32),
            pltpu.VMEM((2 * CH, 16), jnp.float32),
            pltpu.VMEM((2 * CH, 128), jnp.float32),
            pltpu.VMEM((2 * CH, 16), jnp.float32),
            pltpu.VMEM((2 * CH, 128), jnp.float32),
            pltpu.VMEM_SHARED((NP, 128), jnp.float32),
            pltpu.VMEM_SHARED((NP, 16), jnp.float32),
            pltpu.SemaphoreType.DMA,
            pltpu.SemaphoreType.DMA,
            pltpu.SemaphoreType.DMA,
        ],
    )
    return f(src, dst, asrc, adst, h)


def _comb_body(p0_ref, p1_ref, d0_ref, d1_ref, r16_ref, wk_ref, bk_ref, q_ref,
               out_ref, s_ref):
    den = jnp.dot(d0_ref[...] + d1_ref[...], r16_ref[...],
                  preferred_element_type=jnp.float32) + 1e-16
    out = jnp.maximum((p0_ref[...] + p1_ref[...]) / den, 0.0)
    out_ref[...] = out
    t = jnp.tanh(jnp.dot(out, wk_ref[...],
                         preferred_element_type=jnp.float32) + bk_ref[...])

    @pl.when(pl.program_id(0) == 0)
    def _():
        s_ref[0, 0] = 0.0

    s_ref[0, 0] += jnp.sum(t * q_ref[...])


def _comb(p0, p1, d0, d1, r16, wk, bk, qrow):
    return pl.pallas_call(
        _comb_body,
        grid=(N_S // BLK,),
        in_specs=[
            pl.BlockSpec((BLK, 128), lambda i: (i, 0)),
            pl.BlockSpec((BLK, 128), lambda i: (i, 0)),
            pl.BlockSpec((BLK, 16), lambda i: (i, 0)),
            pl.BlockSpec((BLK, 16), lambda i: (i, 0)),
            pl.BlockSpec((16, 128), lambda i: (0, 0)),
            pl.BlockSpec((DH, DH), lambda i: (0, 0)),
            pl.BlockSpec((1, DH), lambda i: (0, 0)),
            pl.BlockSpec((1, DH), lambda i: (0, 0)),
        ],
        out_specs=[
            pl.BlockSpec((BLK, DH), lambda i: (i, 0)),
            pl.BlockSpec((1, 1), lambda i: (0, 0), memory_space=pltpu.SMEM),
        ],
        out_shape=[
            jax.ShapeDtypeStruct((N_S, DH), jnp.float32),
            jax.ShapeDtypeStruct((1, 1), jnp.float32),
        ],
    )(p0, p1, d0, d1, r16, wk, bk, qrow)


def _final_body(ss_ref, rs_ref, s0_ref, s1_ref, wout_ref, bout_ref, o_ref):
    d = (s1_ref[0, 0] - s0_ref[0, 0]) / N_S
    w0 = 1.0 / (1.0 + jnp.exp(jnp.full((1, 128), d, jnp.float32)))
    mix = ss_ref[...] * w0 + rs_ref[...] * (1.0 - w0)
    o_ref[...] = jnp.dot(mix, wout_ref[...],
                         preferred_element_type=jnp.float32) + bout_ref[...]


def _final(out_ss, out_rs, s_ss, s_rs, wout, bout):
    return pl.pallas_call(
        _final_body,
        grid=(N_S // BLK,),
        in_specs=[
            pl.BlockSpec((BLK, DH), lambda i: (i, 0)),
            pl.BlockSpec((BLK, DH), lambda i: (i, 0)),
            pl.BlockSpec(memory_space=pltpu.SMEM),
            pl.BlockSpec(memory_space=pltpu.SMEM),
            pl.BlockSpec((DH, 128), lambda i: (0, 0)),
            pl.BlockSpec((1, 128), lambda i: (0, 0)),
        ],
        out_specs=pl.BlockSpec((BLK, 128), lambda i: (i, 0)),
        out_shape=jax.ShapeDtypeStruct((N_S, 128), jnp.float32),
    )(out_ss, out_rs, s_ss, s_rs, wout, bout)


def _att_mat(att):
    rows = jnp.arange(DH)
    return jnp.zeros((DH, 16), jnp.float32).at[rows, rows // D].set(
        att.reshape(DH))


def kernel(x_SUBJECT, x_REGION, edge_index_SUBJECT__to__SUBJECT,
           edge_index_REGION__in__SUBJECT,
           W_proj_SUBJECT, b_proj_SUBJECT, W_proj_REGION, b_proj_REGION,
           att_src_S2S, att_dst_S2S, att_src_R2S, att_dst_R2S,
           W_k, b_k, q, W_out, b_out):
    ei_ss = edge_index_SUBJECT__to__SUBJECT
    ei_rs = edge_index_REGION__in__SUBJECT

    a_s = jnp.concatenate(
        [_att_mat(att_src_S2S), _att_mat(att_dst_S2S), _att_mat(att_dst_R2S),
         jnp.zeros((DH, 128 - 48), jnp.float32)], axis=1)
    a_r = jnp.concatenate(
        [_att_mat(att_src_R2S), jnp.zeros((DH, 128 - 16), jnp.float32)],
        axis=1)
    cols = jnp.arange(128)
    r16 = jnp.zeros((16, 128), jnp.float32).at[cols // D, cols].set(1.0)
    wout_pad = jnp.zeros((DH, 128), jnp.float32).at[:, :2].set(W_out)
    bout_pad = jnp.zeros((1, 128), jnp.float32).at[0, :2].set(b_out)

    h_S, aS = _proj(x_SUBJECT, W_proj_SUBJECT, b_proj_SUBJECT, a_s)
    h_R, aR = _proj(x_REGION, W_proj_REGION, b_proj_REGION, a_r)

    psum_ss, pden_ss = _edge_sc(ei_ss[0], ei_ss[1],
                                aS[:, 0:16], aS[:, 16:32], h_S)
    psum_rs, pden_rs = _edge_sc(ei_rs[0], ei_rs[1],
                                aR[:, 0:16], aS[:, 32:48], h_R)

    out_ss, s_ss = _comb(psum_ss[:N_S], psum_ss[NP:NP + N_S],
                         pden_ss[:N_S], pden_ss[NP:NP + N_S],
                         r16, W_k, b_k.reshape(1, DH), q.reshape(1, DH))
    out_rs, s_rs = _comb(psum_rs[:N_S], psum_rs[NP:NP + N_S],
                         pden_rs[:N_S], pden_rs[NP:NP + N_S],
                         r16, W_k, b_k.reshape(1, DH), q.reshape(1, DH))

    return _final(out_ss, out_rs, s_ss, s_rs, wout_pad, bout_pad)[:, :2]


# speedup vs baseline: 1.1064x; 1.0005x over previous
import functools

import jax
import jax.numpy as jnp
import numpy as np
from jax import lax
from jax.experimental import pallas as pl
from jax.experimental.pallas import tpu as pltpu
from jax.experimental.pallas import tpu_sc as plsc

N_S = 10000
N_R = 10000
E = 320000
DIN = 128
DH = 128
H = 8
D = 16
NEG_SLOPE = 0.2

NW = 32
EPW = E // NW
CH = 40
NCHUNK = EPW // CH
NP = 10240
RPT = NP // 16
BLK = 400

_DNUMS = lax.GatherDimensionNumbers(
    offset_dims=(), collapsed_slice_dims=(0,), start_index_map=(0,))


def _proj_body(x_ref, w_ref, b_ref, a_ref, h_ref, ha_ref):
    h = jnp.dot(x_ref[...], w_ref[...], preferred_element_type=jnp.float32)
    h = h + b_ref[...]
    h_ref[...] = h
    ha_ref[...] = jnp.dot(h, a_ref[...], preferred_element_type=jnp.float32)


def _proj(x, W, b, A):
    n = x.shape[0]
    return pl.pallas_call(
        _proj_body,
        grid=(n // BLK,),
        in_specs=[
            pl.BlockSpec((BLK, DIN), lambda i: (i, 0)),
            pl.BlockSpec((DIN, DH), lambda i: (0, 0)),
            pl.BlockSpec((1, DH), lambda i: (0, 0)),
            pl.BlockSpec((DH, 128), lambda i: (0, 0)),
        ],
        out_specs=[
            pl.BlockSpec((BLK, DH), lambda i: (i, 0)),
            pl.BlockSpec((BLK, 128), lambda i: (i, 0)),
        ],
        out_shape=[
            jax.ShapeDtypeStruct((n, DH), jnp.float32),
            jax.ShapeDtypeStruct((n, 128), jnp.float32),
        ],
    )(x, W, b.reshape(1, DH), A)


def _edge_sc_body(src_hbm, dst_hbm, asrc_hbm, adst_hbm, h_hbm,
                  psum_hbm, pden_hbm,
                  sidx, didx, as_v, ad_v, h_v, ex_v, msg_v,
                  out_sh, den_sh, sem_i, sem_g, sem_s):
    c = lax.axis_index("c")
    s = lax.axis_index("s")
    wid = s * 2 + c
    base = wid * EPW

    zeros16 = jnp.zeros((16,), jnp.float32)
    lanes = lax.iota(jnp.int32, 16)
    mask = jnp.where(lanes < 8, 1.0, 0.0)
    idx_h = [(lanes * 0 + h).reshape(16, 1) for h in range(8)]

    def zrow(e, carry):
        for j in range(8):
            msg_v[e, pl.ds(16 * j, 16)] = zeros16
        ex_v[e, :] = zeros16
        return carry

    lax.fori_loop(0, CH, zrow, 0)

    def zblk(i, carry):
        off = s * RPT + i * CH
        pltpu.sync_copy(msg_v.at[pl.ds(0, CH)], out_sh.at[pl.ds(off, CH)])
        pltpu.sync_copy(ex_v.at[pl.ds(0, CH)], den_sh.at[pl.ds(off, CH)])
        return carry

    lax.fori_loop(0, RPT // CH, zblk, 0)
    plsc.subcore_barrier()

    def issue_idx(k):
        q = lax.rem(k, 4)
        b = base + k * CH
        pltpu.async_copy(src_hbm.at[pl.ds(b, CH)], sidx.at[q], sem_i)
        pltpu.async_copy(dst_hbm.at[pl.ds(b, CH)], didx.at[q], sem_i)

    def wait_idx():
        pltpu.make_async_copy(
            src_hbm.at[pl.ds(0, CH)], sidx.at[0], sem_i).wait()
        pltpu.make_async_copy(
            dst_hbm.at[pl.ds(0, CH)], didx.at[0], sem_i).wait()

    def issue_gathers(k):
        q = lax.rem(k, 4)
        p = lax.rem(k, 2)
        o = p * CH
        pltpu.async_copy(asrc_hbm.at[sidx.at[q]],
                         as_v.at[pl.ds(o, CH)], sem_g)
        pltpu.async_copy(adst_hbm.at[didx.at[q]],
                         ad_v.at[pl.ds(o, CH)], sem_g)
        pltpu.async_copy(h_hbm.at[sidx.at[q]],
                         h_v.at[pl.ds(o, CH)], sem_g)

    def wait_gathers():
        pltpu.make_async_copy(
            asrc_hbm.at[pl.ds(0, CH)], as_v.at[pl.ds(0, CH)], sem_g).wait()
        pltpu.make_async_copy(
            adst_hbm.at[pl.ds(0, CH)], ad_v.at[pl.ds(0, CH)], sem_g).wait()
        pltpu.make_async_copy(
            h_hbm.at[pl.ds(0, CH)], h_v.at[pl.ds(0, CH)], sem_g).wait()

    def wait_scatters():
        pltpu.make_async_copy(
            asrc_hbm.at[pl.ds(0, CH)], ex_v.at[pl.ds(0, CH)], sem_s).wait()
        pltpu.make_async_copy(
            h_hbm.at[pl.ds(0, CH)], msg_v.at[pl.ds(0, CH)], sem_s).wait()

    issue_idx(0)
    wait_idx()
    issue_gathers(0)
    issue_idx(1)

    def chunk(k, carry):
        p = lax.rem(k, 2)
        q = lax.rem(k, 4)
        o = p * CH
        wait_gathers()

        @pl.when(k >= 2)
        def _():
            wait_scatters()

        @pl.when(k + 1 < NCHUNK)
        def _():
            wait_idx()
            issue_gathers(k + 1)

        @pl.when(k + 2 < NCHUNK)
        def _():
            issue_idx(k + 2)

        @plsc.parallel_loop(0, CH, 1, unroll=8)
        def edge(e):
            eo = o + e
            alpha = as_v[eo, :] + ad_v[eo, :]
            alpha = jnp.where(alpha > 0, alpha, NEG_SLOPE * alpha)
            ex = jnp.exp(alpha) * mask
            ex_v[eo, :] = ex
            for h in range(8):
                spl = lax.gather(ex, idx_h[h], _DNUMS, (1,),
                                 mode=lax.GatherScatterMode.PROMISE_IN_BOUNDS)
                msg_v[eo, pl.ds(16 * h, 16)] = (
                    h_v[eo, pl.ds(16 * h, 16)] * spl)
        pltpu.async_copy(ex_v.at[pl.ds(o, CH)],
                         den_sh.at[didx.at[q]], sem_s, add=True)
        pltpu.async_copy(msg_v.at[pl.ds(o, CH)],
                         out_sh.at[didx.at[q]], sem_s, add=True)
        return carry

    lax.fori_loop(0, NCHUNK, chunk, 0)
    wait_scatters()
    wait_scatters()
    plsc.subcore_barrier()

    off = c * NP + s * RPT
    pltpu.sync_copy(out_sh.at[pl.ds(s * RPT, RPT)], psum_hbm.at[pl.ds(off, RPT)])
    pltpu.sync_copy(den_sh.at[pl.ds(s * RPT, RPT)], pden_hbm.at[pl.ds(off, RPT)])


def _edge_sc(src, dst, asrc, adst, h):
    mesh = plsc.VectorSubcoreMesh(core_axis_name="c", subcore_axis_name="s")
    f = pl.kernel(
        _edge_sc_body,
        out_type=[
            jax.ShapeDtypeStruct((2 * NP, 128), jnp.float32),
            jax.ShapeDtypeStruct((2 * NP, 16), jnp.float32),
        ],
        mesh=mesh,
        compiler_params=pltpu.CompilerParams(
            needs_layout_passes=False, use_tc_tiling_on_sc=False),
        scratch_types=[
            pltpu.VMEM((4, CH), jnp.int32),
            pltpu.VMEM((4, CH), jnp.int32),
            pltpu.VMEM((2 * CH, 16), jnp.float32),
            pltpu.VMEM((2 * CH, 16), jnp.float32),
            pltpu.VMEM((2 * CH, 128), jnp.float32),
            pltpu.VMEM((2 * CH, 16), jnp.float32),
            pltpu.VMEM((2 * CH, 128), jnp.float32),
            pltpu.VMEM_SHARED((NP, 128), jnp.float32),
            pltpu.VMEM_SHARED((NP, 16), jnp.float32),
            pltpu.SemaphoreType.DMA,
            pltpu.SemaphoreType.DMA,
            pltpu.SemaphoreType.DMA,
        ],
    )
    return f(src, dst, asrc, adst, h)


def _comb_body(p0_ref, p1_ref, d0_ref, d1_ref, r16_ref, wk_ref, bk_ref, q_ref,
               out_ref, s_ref):
    den = jnp.dot(d0_ref[...] + d1_ref[...], r16_ref[...],
                  preferred_element_type=jnp.float32) + 1e-16
    out = jnp.maximum((p0_ref[...] + p1_ref[...]) / den, 0.0)
    out_ref[...] = out
    t = jnp.tanh(jnp.dot(out, wk_ref[...],
                         preferred_element_type=jnp.float32) + bk_ref[...])

    @pl.when(pl.program_id(0) == 0)
    def _():
        s_ref[0, 0] = 0.0

    s_ref[0, 0] += jnp.sum(t * q_ref[...])


def _comb(p0, p1, d0, d1, r16, wk, bk, qrow):
    return pl.pallas_call(
        _comb_body,
        grid=(N_S // BLK,),
        in_specs=[
            pl.BlockSpec((BLK, 128), lambda i: (i, 0)),
            pl.BlockSpec((BLK, 128), lambda i: (i, 0)),
            pl.BlockSpec((BLK, 16), lambda i: (i, 0)),
            pl.BlockSpec((BLK, 16), lambda i: (i, 0)),
            pl.BlockSpec((16, 128), lambda i: (0, 0)),
            pl.BlockSpec((DH, DH), lambda i: (0, 0)),
            pl.BlockSpec((1, DH), lambda i: (0, 0)),
            pl.BlockSpec((1, DH), lambda i: (0, 0)),
        ],
        out_specs=[
            pl.BlockSpec((BLK, DH), lambda i: (i, 0)),
            pl.BlockSpec((1, 1), lambda i: (0, 0), memory_space=pltpu.SMEM),
        ],
        out_shape=[
            jax.ShapeDtypeStruct((N_S, DH), jnp.float32),
            jax.ShapeDtypeStruct((1, 1), jnp.float32),
        ],
    )(p0, p1, d0, d1, r16, wk, bk, qrow)


def _final_body(ss_ref, rs_ref, s0_ref, s1_ref, wout_ref, bout_ref, o_ref):
    d = (s1_ref[0, 0] - s0_ref[0, 0]) / N_S
    w0 = 1.0 / (1.0 + jnp.exp(jnp.full((1, 128), d, jnp.float32)))
    mix = ss_ref[...] * w0 + rs_ref[...] * (1.0 - w0)
    o_ref[...] = jnp.dot(mix, wout_ref[...],
                         preferred_element_type=jnp.float32) + bout_ref[...]


def _final(out_ss, out_rs, s_ss, s_rs, wout, bout):
    return pl.pallas_call(
        _final_body,
        grid=(N_S // BLK,),
        in_specs=[
            pl.BlockSpec((BLK, DH), lambda i: (i, 0)),
            pl.BlockSpec((BLK, DH), lambda i: (i, 0)),
            pl.BlockSpec(memory_space=pltpu.SMEM),
            pl.BlockSpec(memory_space=pltpu.SMEM),
            pl.BlockSpec((DH, 128), lambda i: (0, 0)),
            pl.BlockSpec((1, 128), lambda i: (0, 0)),
        ],
        out_specs=pl.BlockSpec((BLK, 128), lambda i: (i, 0)),
        out_shape=jax.ShapeDtypeStruct((N_S, 128), jnp.float32),
    )(out_ss, out_rs, s_ss, s_rs, wout, bout)


def _att_mat(att):
    rows = jnp.arange(DH)
    return jnp.zeros((DH, 16), jnp.float32).at[rows, rows // D].set(
        att.reshape(DH))


def kernel(x_SUBJECT, x_REGION, edge_index_SUBJECT__to__SUBJECT,
           edge_index_REGION__in__SUBJECT,
           W_proj_SUBJECT, b_proj_SUBJECT, W_proj_REGION, b_proj_REGION,
           att_src_S2S, att_dst_S2S, att_src_R2S, att_dst_R2S,
           W_k, b_k, q, W_out, b_out):
    ei_ss = edge_index_SUBJECT__to__SUBJECT
    ei_rs = edge_index_REGION__in__SUBJECT

    a_s = jnp.concatenate(
        [_att_mat(att_src_S2S), _att_mat(att_dst_S2S), _att_mat(att_dst_R2S),
         jnp.zeros((DH, 128 - 48), jnp.float32)], axis=1)
    a_r = jnp.concatenate(
        [_att_mat(att_src_R2S), jnp.zeros((DH, 128 - 16), jnp.float32)],
        axis=1)
    cols = jnp.arange(128)
    r16 = jnp.zeros((16, 128), jnp.float32).at[cols // D, cols].set(1.0)
    wout_pad = jnp.zeros((DH, 128), jnp.float32).at[:, :2].set(W_out)
    bout_pad = jnp.zeros((1, 128), jnp.float32).at[0, :2].set(b_out)

    h_S, aS = _proj(x_SUBJECT, W_proj_SUBJECT, b_proj_SUBJECT, a_s)
    h_R, aR = _proj(x_REGION, W_proj_REGION, b_proj_REGION, a_r)

    psum_ss, pden_ss = _edge_sc(ei_ss[0], ei_ss[1],
                                aS[:, 0:16], aS[:, 16:32], h_S)
    psum_rs, pden_rs = _edge_sc(ei_rs[0], ei_rs[1],
                                aR[:, 0:16], aS[:, 32:48], h_R)

    out_ss, s_ss = _comb(psum_ss[:N_S], psum_ss[NP:NP + N_S],
                         pden_ss[:N_S], pden_ss[NP:NP + N_S],
                         r16, W_k, b_k.reshape(1, DH), q.reshape(1, DH))
    out_rs, s_rs = _comb(psum_rs[:N_S], psum_rs[NP:NP + N_S],
                         pden_rs[:N_S], pden_rs[NP:NP + N_S],
                         r16, W_k, b_k.reshape(1, DH), q.reshape(1, DH))

    return _final(out_ss, out_rs, s_ss, s_rs, wout_pad, bout_pad)[:, :2]
